# Initial kernel scaffold; baseline (speedup 1.0000x reference)
import jax
import jax.numpy as jnp
from jax.experimental import pallas as pl


def kernel(x, edge_index, W_rel, W_root):
    raise NotImplementedError("write your pallas kernel here")


# speedup vs baseline: 4.6632x; 4.6632x over previous
import functools

import jax
import jax.numpy as jnp
from jax import lax
from jax.experimental import pallas as pl
from jax.experimental.pallas import tpu as pltpu
from jax.experimental.pallas import tpu_sc as plsc

NC = 2
NS = 16
NW = NC * NS
LANES = 16
CHUNK = 128
N_PAD = 10240


def _sc_agg(x, src3, dst3, n_chunks):
    cin = x.shape[1]
    rows_per_sub = N_PAD // NS

    mesh = plsc.VectorSubcoreMesh(core_axis_name="c", subcore_axis_name="s")

    @functools.partial(
        pl.kernel,
        out_type=jax.ShapeDtypeStruct((NC, N_PAD, cin), jnp.float32),
        mesh=mesh,
        scratch_types=[
            pltpu.VMEM((n_chunks, CHUNK), jnp.int32),
            pltpu.VMEM((n_chunks, CHUNK), jnp.int32),
            pltpu.VMEM((CHUNK, cin), jnp.float32),
            pltpu.VMEM_SHARED((N_PAD, cin), jnp.float32),
            pltpu.SemaphoreType.DMA,
        ],
    )
    def body(x_hbm, src_hbm, dst_hbm, out_hbm, src_v, dst_v, rows_v, agg_sh,
             sem):
        c = lax.axis_index("c")
        s = lax.axis_index("s")
        wid = c * NS + s

        def zrow(i, _):
            def zcol(k, __):
                rows_v[i, pl.ds(k * LANES, LANES)] = jnp.zeros(
                    (LANES,), jnp.float32)
                return 0
            return lax.fori_loop(0, cin // LANES, zcol, 0)
        lax.fori_loop(0, CHUNK, zrow, 0)

        base = s * rows_per_sub
        for m in range(rows_per_sub // CHUNK):
            pltpu.sync_copy(rows_v, agg_sh.at[pl.ds(base + m * CHUNK, CHUNK)])
        plsc.subcore_barrier()

        pltpu.sync_copy(src_hbm.at[wid], src_v)
        pltpu.sync_copy(dst_hbm.at[wid], dst_v)

        def step(j, _):
            pltpu.async_copy(x_hbm.at[src_v.at[j]], rows_v, sem).wait()
            pltpu.sync_copy(rows_v, agg_sh.at[dst_v.at[j]], add=True)
            return 0
        lax.fori_loop(0, n_chunks, step, 0)

        plsc.subcore_barrier()
        pltpu.sync_copy(agg_sh.at[pl.ds(base, rows_per_sub)],
                        out_hbm.at[c, pl.ds(base, rows_per_sub)])

    return body(x, src3, dst3)


def _tc_combine(a0, a1, x, wr_t, wo_t):
    n, cin = x.shape
    cout = wr_t.shape[1]
    bm = 1000

    def body(a0_ref, a1_ref, x_ref, wr_ref, wo_ref, o_ref):
        agg = a0_ref[...] + a1_ref[...]
        acc = jnp.dot(agg, wr_ref[...], preferred_element_type=jnp.float32)
        acc = acc + jnp.dot(x_ref[...], wo_ref[...],
                            preferred_element_type=jnp.float32)
        o_ref[...] = jnp.maximum(acc, 0.0)

    return pl.pallas_call(
        body,
        grid=(n // bm,),
        in_specs=[
            pl.BlockSpec((bm, cin), lambda i: (i, 0)),
            pl.BlockSpec((bm, cin), lambda i: (i, 0)),
            pl.BlockSpec((bm, cin), lambda i: (i, 0)),
            pl.BlockSpec((cin, cout), lambda i: (0, 0)),
            pl.BlockSpec((cin, cout), lambda i: (0, 0)),
        ],
        out_specs=pl.BlockSpec((bm, cout), lambda i: (i, 0)),
        out_shape=jax.ShapeDtypeStruct((n, cout), jnp.float32),
    )(a0, a1, x, wr_t, wo_t)


def kernel(x, edge_index, W_rel, W_root):
    n = x.shape[0]
    src = edge_index[0].astype(jnp.int32)
    dst = edge_index[1].astype(jnp.int32)
    e = src.shape[0]

    n_chunks = -(-e // (NW * CHUNK))
    e_pad = NW * n_chunks * CHUNK
    pad = e_pad - e
    src_p = jnp.concatenate([src, jnp.zeros((pad,), jnp.int32)])
    dst_p = jnp.concatenate([dst, jnp.full((pad,), n, jnp.int32)])
    src3 = src_p.reshape(NW, n_chunks, CHUNK)
    dst3 = dst_p.reshape(NW, n_chunks, CHUNK)

    parts = _sc_agg(x, src3, dst3, n_chunks)
    return _tc_combine(parts[0, :n], parts[1, :n], x, W_rel.T, W_root.T)
